# Initial kernel scaffold; baseline (speedup 1.0000x reference)
import functools

import jax
import jax.numpy as jnp
from jax import lax
from jax.experimental import pallas as pl
from jax.experimental.pallas import tpu as pltpu
from jax.experimental.pallas import tpu_sc as plsc

VOCAB = 1000
N_EMBED = 16
B = 1024
T = 20
TOT = B * T

NUM_CORES = 2
NUM_SUBCORES = 16
NW = NUM_CORES * NUM_SUBCORES
ROWS_PER_W = TOT // NW
CHUNK = 64
N_CHUNKS = ROWS_PER_W // CHUNK


def _logits_table_body(t_ref, w_ref, b_ref, o_ref):
    o_ref[...] = (
        jnp.dot(t_ref[...], w_ref[...], preferred_element_type=jnp.float32)
        + b_ref[...]
    )


def _logits_table(table, w, b):
    return pl.pallas_call(
        _logits_table_body,
        out_shape=jax.ShapeDtypeStruct((VOCAB, VOCAB), jnp.float32),
    )(table, w, b.reshape(1, VOCAB))


def _gather_body(lt_hbm, idx_hbm, out_hbm, idx_v, buf0, buf1, sem0, sem1):
    wid = lax.axis_index("s") * NUM_CORES + lax.axis_index("c")
    base = wid * ROWS_PER_W
    pltpu.sync_copy(idx_hbm.at[pl.ds(base, ROWS_PER_W)], idx_v)

    bufs = (buf0, buf1)
    sems = (sem0, sem1)

    pltpu.async_copy(lt_hbm.at[idx_v.at[pl.ds(0, CHUNK)]], buf0, sem0)

    def step(i, _):
        slot = lax.rem(i, 2)

        def do(s):
            cur_buf, cur_sem = bufs[s], sems[s]
            nxt_buf, nxt_sem = bufs[1 - s], sems[1 - s]
            pltpu.make_async_copy(
                lt_hbm.at[idx_v.at[pl.ds(i * CHUNK, CHUNK)]], cur_buf, cur_sem
            ).wait()

            @pl.when(i + 1 < N_CHUNKS)
            def _():
                pltpu.async_copy(
                    lt_hbm.at[idx_v.at[pl.ds((i + 1) * CHUNK, CHUNK)]],
                    nxt_buf,
                    nxt_sem,
                )

            pltpu.sync_copy(cur_buf, out_hbm.at[pl.ds(base + i * CHUNK, CHUNK)])

        lax.cond(slot == 0, lambda: do(0), lambda: do(1))
        return 0

    lax.fori_loop(0, N_CHUNKS, step, 0)


def _gather(lt, idx_flat):
    mesh = plsc.VectorSubcoreMesh(core_axis_name="c", subcore_axis_name="s")
    k = functools.partial(
        pl.kernel,
        out_type=jax.ShapeDtypeStruct((TOT, VOCAB), jnp.float32),
        mesh=mesh,
        scratch_types=[
            pltpu.VMEM((ROWS_PER_W,), jnp.int32),
            pltpu.VMEM((CHUNK, VOCAB), jnp.float32),
            pltpu.VMEM((CHUNK, VOCAB), jnp.float32),
            pltpu.SemaphoreType.DMA,
            pltpu.SemaphoreType.DMA,
        ],
    )(_gather_body)
    return k(lt, idx_flat)


def kernel(idx, token_embedding_table, lm_head_w, lm_head_b):
    lt = _logits_table(token_embedding_table, lm_head_w, lm_head_b)
    idx_flat = idx.reshape(-1).astype(jnp.int32)
    return _gather(lt, idx_flat)


# speedup vs baseline: 1.4013x; 1.4013x over previous
import functools

import jax
import jax.numpy as jnp
from jax import lax
from jax.experimental import pallas as pl
from jax.experimental.pallas import tpu as pltpu
from jax.experimental.pallas import tpu_sc as plsc

VOCAB = 1000
N_EMBED = 16
B = 1024
T = 20
TOT = B * T

NUM_CORES = 2
NUM_SUBCORES = 16
NW = NUM_CORES * NUM_SUBCORES
ROWS_PER_W = TOT // NW
CHUNK = 64
N_CHUNKS = ROWS_PER_W // CHUNK


def _logits_table_body(t_ref, w_ref, b_ref, o_ref):
    o_ref[...] = (
        jnp.dot(t_ref[...], w_ref[...], preferred_element_type=jnp.float32)
        + b_ref[...]
    )


def _logits_table(table, w, b):
    return pl.pallas_call(
        _logits_table_body,
        out_shape=jax.ShapeDtypeStruct((VOCAB, VOCAB), jnp.float32),
    )(table, w, b.reshape(1, VOCAB))


def _gather_body(lt_hbm, idx_hbm, out_hbm, idx_v, buf0, buf1, sem0, sem1):
    wid = lax.axis_index("s") * NUM_CORES + lax.axis_index("c")
    base = wid * ROWS_PER_W
    pltpu.sync_copy(idx_hbm.at[pl.ds(base, ROWS_PER_W)], idx_v)

    bufs = (buf0, buf1)
    sems = (sem0, sem1)

    pltpu.async_copy(lt_hbm.at[idx_v.at[pl.ds(0, CHUNK)]], buf0, sem0)

    def step(i, _):
        slot = lax.rem(i, 2)

        def do(s):
            cur_buf, cur_sem = bufs[s], sems[s]
            nxt_buf, nxt_sem = bufs[1 - s], sems[1 - s]
            pltpu.make_async_copy(
                lt_hbm.at[idx_v.at[pl.ds(i * CHUNK, CHUNK)]], cur_buf, cur_sem
            ).wait()

            @pl.when(i + 1 < N_CHUNKS)
            def _():
                pltpu.async_copy(
                    lt_hbm.at[idx_v.at[pl.ds((i + 1) * CHUNK, CHUNK)]],
                    nxt_buf,
                    nxt_sem,
                )

            pltpu.sync_copy(cur_buf, out_hbm.at[pl.ds(base + i * CHUNK, CHUNK)])

        lax.cond(slot == 0, lambda: do(0), lambda: do(1))
        return 0

    lax.fori_loop(0, N_CHUNKS, step, 0)


def _gather(lt, idx_flat):
    mesh = plsc.VectorSubcoreMesh(core_axis_name="c", subcore_axis_name="s")
    k = functools.partial(
        pl.kernel,
        out_type=jax.ShapeDtypeStruct((TOT, VOCAB), jnp.float32),
        mesh=mesh,
        scratch_types=[
            pltpu.VMEM((ROWS_PER_W,), jnp.int32),
            pltpu.VMEM((CHUNK, VOCAB), jnp.float32),
            pltpu.VMEM((CHUNK, VOCAB), jnp.float32),
            pltpu.SemaphoreType.DMA,
            pltpu.SemaphoreType.DMA,
        ],
        compiler_params=pltpu.CompilerParams(use_tc_tiling_on_sc=False),
    )(_gather_body)
    return k(lt, idx_flat)


def kernel(idx, token_embedding_table, lm_head_w, lm_head_b):
    lt = _logits_table(token_embedding_table, lm_head_w, lm_head_b)
    idx_flat = idx.reshape(-1).astype(jnp.int32)
    return _gather(lt, idx_flat)
